# baseline (speedup 1.0000x reference)
import functools

import jax
import jax.numpy as jnp
import numpy as np
from jax import lax
from jax.experimental import pallas as pl
from jax.experimental.pallas import tpu as pltpu
from jax.experimental.pallas import tpu_sc as plsc


def _build_sc_kernel(b, h, w, d):
    mesh = plsc.VectorSubcoreMesh(core_axis_name="c", subcore_axis_name="s")

    @functools.partial(
        pl.kernel,
        mesh=mesh,
        out_type=jax.ShapeDtypeStruct((b, h * w, 2 * d), jnp.float32),
        scratch_types=[
            pltpu.VMEM((w, d), jnp.float32),
            pltpu.VMEM((d,), jnp.float32),
            pltpu.VMEM((w, 2 * d), jnp.float32),
            pltpu.SemaphoreType.DMA,
        ],
    )
    def sc_kernel(row_hbm, col_hbm, out_hbm, colv, rowv, buf, sem):
        cid = lax.axis_index("c")
        sid = lax.axis_index("s")
        i = sid * 2 + cid

        pltpu.sync_copy(col_hbm, colv)
        pltpu.sync_copy(row_hbm.at[i], rowv)

        for c in range(d // 16):
            v = rowv[pl.ds(c * 16, 16)]
            for j in range(w):
                buf[j, pl.ds(d + c * 16, 16)] = v
        for j in range(w):
            for c in range(d // 16):
                buf[j, pl.ds(c * 16, 16)] = colv[j, pl.ds(c * 16, 16)]

        copies = [
            pltpu.async_copy(buf, out_hbm.at[bb, pl.ds(i * w, w)], sem)
            for bb in range(b)
        ]
        for cp in copies:
            cp.wait()

    return sc_kernel


def kernel(x, row_embed, col_embed):
    b = x.shape[0]
    hw = x.shape[1]
    h = w = int(np.sqrt(hw))
    d = row_embed.shape[1]
    row32 = row_embed[:h]
    col32 = col_embed[:w]
    return _build_sc_kernel(b, h, w, d)(row32, col32)
